# baseline (speedup 1.0000x reference)
import functools

import jax
import jax.numpy as jnp
from jax import lax
from jax.experimental import pallas as pl
from jax.experimental.pallas import tpu as pltpu
from jax.experimental.pallas import tpu_sc as plsc

N = 10000
E = 320000
D = 128

NT = 16
EPT = E // NT
B = 80
NCHUNK = EPT // B
NP = 10240
RPT = NP // NT

BR = 400
G = N // BR


def _phase_a_body(hid_ref, winT_ref, bin_ref, woutT_ref, bout_ref,
                  whhT_ref, bhh_ref, hin_ref, hout_ref, gh_ref):
    x = hid_ref[...]
    hin_ref[...] = jnp.dot(x, winT_ref[...],
                           preferred_element_type=jnp.float32) + bin_ref[...]
    hout_ref[...] = jnp.dot(x, woutT_ref[...],
                            preferred_element_type=jnp.float32) + bout_ref[...]
    gh_ref[...] = jnp.dot(x, whhT_ref[...],
                          preferred_element_type=jnp.float32) + bhh_ref[...]


def _phase_a(hidden, WinT, b_in, WoutT, b_out, WhhT, b_hh):
    full = lambda r, c: pl.BlockSpec((r, c), lambda i: (0, 0))
    row = lambda c: pl.BlockSpec((BR, c), lambda i: (i, 0))
    return pl.pallas_call(
        _phase_a_body,
        grid=(G,),
        in_specs=[row(D), full(D, D), full(1, D), full(D, D), full(1, D),
                  full(D, 3 * D), full(1, 3 * D)],
        out_specs=[row(D), row(D), row(3 * D)],
        out_shape=[jax.ShapeDtypeStruct((N, D), jnp.float32),
                   jax.ShapeDtypeStruct((N, D), jnp.float32),
                   jax.ShapeDtypeStruct((N, 3 * D), jnp.float32)],
    )(hidden, WinT, b_in, WoutT, b_out, WhhT, b_hh)


def _phase_c_body(ain_ref, aout_ref, hid_ref, gh_ref, wa_ref, wb_ref,
                  bih_ref, hy_ref):
    gi = (jnp.dot(ain_ref[...], wa_ref[...],
                  preferred_element_type=jnp.float32)
          + jnp.dot(aout_ref[...], wb_ref[...],
                    preferred_element_type=jnp.float32)
          + bih_ref[...])
    gh = gh_ref[...]
    i_r, i_i, i_n = gi[:, :D], gi[:, D:2 * D], gi[:, 2 * D:]
    h_r, h_i, h_n = gh[:, :D], gh[:, D:2 * D], gh[:, 2 * D:]
    reset_gate = jax.nn.sigmoid(i_r + h_r)
    input_gate = jax.nn.sigmoid(i_i + h_i)
    new_gate = jnp.tanh(i_n + reset_gate * h_n)
    hid = hid_ref[...]
    hy_ref[...] = (1.0 - input_gate) * hid + input_gate * new_gate


def _phase_c(agg_in, agg_out, hidden, gh, Wa, Wb, b_ih):
    full = lambda r, c: pl.BlockSpec((r, c), lambda i: (0, 0))
    row = lambda c: pl.BlockSpec((BR, c), lambda i: (i, 0))
    return pl.pallas_call(
        _phase_c_body,
        grid=(G,),
        in_specs=[row(D), row(D), row(D), row(3 * D),
                  full(D, 3 * D), full(D, 3 * D), full(1, 3 * D)],
        out_specs=row(D),
        out_shape=jax.ShapeDtypeStruct((N, D), jnp.float32),
    )(agg_in, agg_out, hidden, gh, Wa, Wb, b_ih)


def _sc_conv_kernel(h_hbm, src_hbm, dst_hbm, w_hbm, out_hbm,
                    srcb0, srcb1, dstb0, dstb1, wb0, wb1, rows0, rows1,
                    acc, gsem0, gsem1, isem0, isem1):
    sid = lax.axis_index("s")

    def zrow(i, c):
        for k in range(D // 16):
            rows0[i, pl.ds(k * 16, 16)] = jnp.zeros((16,), jnp.float32)
        return c
    lax.fori_loop(0, B, zrow, 0)

    def zcopy(q, c):
        pltpu.sync_copy(rows0, acc.at[pl.ds(sid * RPT + q * B, B)])
        return c
    lax.fori_loop(0, RPT // B, zcopy, 0)
    plsc.subcore_barrier()

    srcb = (srcb0, srcb1)
    dstb = (dstb0, dstb1)
    wb = (wb0, wb1)
    rows = (rows0, rows1)
    gsem = (gsem0, gsem1)
    isem = (isem0, isem1)
    tb = sid * NCHUNK * B

    def idx_issue(j, p):
        base = tb + j * B
        pltpu.async_copy(src_hbm.at[pl.ds(base, B)], srcb[p], isem[p])
        pltpu.async_copy(dst_hbm.at[pl.ds(base, B)], dstb[p], isem[p])
        pltpu.async_copy(w_hbm.at[pl.ds(base, B)], wb[p], isem[p])

    def idx_wait(j, p):
        base = tb + j * B
        pltpu.make_async_copy(src_hbm.at[pl.ds(base, B)], srcb[p],
                              isem[p]).wait()
        pltpu.make_async_copy(dst_hbm.at[pl.ds(base, B)], dstb[p],
                              isem[p]).wait()
        pltpu.make_async_copy(w_hbm.at[pl.ds(base, B)], wb[p],
                              isem[p]).wait()

    def process(j, buf, pf_gather, pf_idx):
        if pf_gather:
            idx_wait(j + 1, 1 - buf)
            pltpu.async_copy(h_hbm.at[srcb[1 - buf]], rows[1 - buf],
                             gsem[1 - buf])
        pltpu.make_async_copy(h_hbm.at[srcb[buf]], rows[buf],
                              gsem[buf]).wait()

        def gbody(g, c2):
            wv = wb[buf][pl.ds(g * 16, 16)]
            for l in range(16):
                ws = jnp.broadcast_to(wv[l:l + 1], (16,))
                r = g * 16 + l
                for k in range(D // 16):
                    sl = pl.ds(k * 16, 16)
                    rows[buf][r, sl] = rows[buf][r, sl] * ws
            return c2
        lax.fori_loop(0, B // 16, gbody, 0)

        pltpu.sync_copy(rows[buf], acc.at[dstb[buf]], add=True)

        if pf_idx:
            idx_issue(j + 2, buf)

    pltpu.sync_copy(src_hbm.at[pl.ds(tb, B)], srcb0)
    pltpu.sync_copy(dst_hbm.at[pl.ds(tb, B)], dstb0)
    pltpu.sync_copy(w_hbm.at[pl.ds(tb, B)], wb0)
    pltpu.async_copy(h_hbm.at[srcb0], rows0, gsem0)
    idx_issue(1, 1)

    def outer(j2, c):
        j = j2 * 2
        process(j, 0, True, True)
        process(j + 1, 1, True, True)
        return c
    lax.fori_loop(0, (NCHUNK - 2) // 2, outer, 0)
    process(NCHUNK - 2, 0, True, False)
    process(NCHUNK - 1, 1, False, False)

    plsc.subcore_barrier()
    pltpu.sync_copy(acc.at[pl.ds(sid * RPT, RPT)],
                    out_hbm.at[pl.ds(sid * RPT, RPT)])


_sc_conv = functools.partial(
    pl.kernel,
    out_type=jax.ShapeDtypeStruct((NP, D), jnp.float32),
    mesh=plsc.VectorSubcoreMesh(core_axis_name="c", subcore_axis_name="s",
                                num_cores=1),
    scratch_types=[
        pltpu.VMEM((B,), jnp.int32),
        pltpu.VMEM((B,), jnp.int32),
        pltpu.VMEM((B,), jnp.int32),
        pltpu.VMEM((B,), jnp.int32),
        pltpu.VMEM((B,), jnp.float32),
        pltpu.VMEM((B,), jnp.float32),
        pltpu.VMEM((B, D), jnp.float32),
        pltpu.VMEM((B, D), jnp.float32),
        pltpu.VMEM_SHARED((NP, D), jnp.float32),
        pltpu.SemaphoreType.DMA,
        pltpu.SemaphoreType.DMA,
        pltpu.SemaphoreType.DMA,
        pltpu.SemaphoreType.DMA,
    ],
)(_sc_conv_kernel)


def kernel(hidden, in_edge_index, in_edge_weight, out_edge_index,
           out_edge_weight, W_in, b_in, W_out, b_out, W_ih, b_ih, W_hh, b_hh):
    hidden = hidden.astype(jnp.float32)

    h_in, h_out, gh = _phase_a(
        hidden, W_in.T, b_in.reshape(1, D), W_out.T, b_out.reshape(1, D),
        W_hh.T, b_hh.reshape(1, 3 * D))

    def shape_edges(ei, ew):
        return (ei[0].astype(jnp.int32), ei[1].astype(jnp.int32),
                ew.astype(jnp.float32))

    src_i, dst_i, w_i = shape_edges(in_edge_index, in_edge_weight)
    src_o, dst_o, w_o = shape_edges(out_edge_index, out_edge_weight)

    agg_in = _sc_conv(h_in, src_i, dst_i, w_i)
    agg_out = _sc_conv(h_out, src_o, dst_o, w_o)

    W_ihT = W_ih.T
    return _phase_c(agg_in[:N], agg_out[:N], hidden, gh,
                    W_ihT[:D], W_ihT[D:], b_ih.reshape(1, 3 * D))


# speedup vs baseline: 1.2336x; 1.2336x over previous
import functools

import jax
import jax.numpy as jnp
from jax import lax
from jax.experimental import pallas as pl
from jax.experimental.pallas import tpu as pltpu
from jax.experimental.pallas import tpu_sc as plsc

N = 10000
E = 320000
D = 128

NT = 16
EPT = E // NT
B = 80
NCHUNK = EPT // B
NP = 10240
RPT = NP // NT

BR = 400
G = N // BR


def _phase_a_body(hid_ref, winT_ref, bin_ref, woutT_ref, bout_ref,
                  whhT_ref, bhh_ref, hin_ref, hout_ref, gh_ref):
    x = hid_ref[...]
    hin_ref[...] = jnp.dot(x, winT_ref[...],
                           preferred_element_type=jnp.float32) + bin_ref[...]
    hout_ref[...] = jnp.dot(x, woutT_ref[...],
                            preferred_element_type=jnp.float32) + bout_ref[...]
    gh_ref[...] = jnp.dot(x, whhT_ref[...],
                          preferred_element_type=jnp.float32) + bhh_ref[...]


def _phase_a(hidden, WinT, b_in, WoutT, b_out, WhhT, b_hh):
    full = lambda r, c: pl.BlockSpec((r, c), lambda i: (0, 0))
    row = lambda c: pl.BlockSpec((BR, c), lambda i: (i, 0))
    return pl.pallas_call(
        _phase_a_body,
        grid=(G,),
        in_specs=[row(D), full(D, D), full(1, D), full(D, D), full(1, D),
                  full(D, 3 * D), full(1, 3 * D)],
        out_specs=[row(D), row(D), row(3 * D)],
        out_shape=[jax.ShapeDtypeStruct((N, D), jnp.float32),
                   jax.ShapeDtypeStruct((N, D), jnp.float32),
                   jax.ShapeDtypeStruct((N, 3 * D), jnp.float32)],
    )(hidden, WinT, b_in, WoutT, b_out, WhhT, b_hh)


def _phase_c_body(ain_ref, aout_ref, hid_ref, gh_ref, wa_ref, wb_ref,
                  bih_ref, hy_ref):
    gi = (jnp.dot(ain_ref[...], wa_ref[...],
                  preferred_element_type=jnp.float32)
          + jnp.dot(aout_ref[...], wb_ref[...],
                    preferred_element_type=jnp.float32)
          + bih_ref[...])
    gh = gh_ref[...]
    i_r, i_i, i_n = gi[:, :D], gi[:, D:2 * D], gi[:, 2 * D:]
    h_r, h_i, h_n = gh[:, :D], gh[:, D:2 * D], gh[:, 2 * D:]
    reset_gate = jax.nn.sigmoid(i_r + h_r)
    input_gate = jax.nn.sigmoid(i_i + h_i)
    new_gate = jnp.tanh(i_n + reset_gate * h_n)
    hid = hid_ref[...]
    hy_ref[...] = (1.0 - input_gate) * hid + input_gate * new_gate


def _phase_c(agg_in, agg_out, hidden, gh, Wa, Wb, b_ih):
    full = lambda r, c: pl.BlockSpec((r, c), lambda i: (0, 0))
    row = lambda c: pl.BlockSpec((BR, c), lambda i: (i, 0))
    return pl.pallas_call(
        _phase_c_body,
        grid=(G,),
        in_specs=[row(D), row(D), row(D), row(3 * D),
                  full(D, 3 * D), full(D, 3 * D), full(1, 3 * D)],
        out_specs=row(D),
        out_shape=jax.ShapeDtypeStruct((N, D), jnp.float32),
    )(agg_in, agg_out, hidden, gh, Wa, Wb, b_ih)


def _sc_conv_kernel(h_hbm, src_hbm, dst_hbm, w_hbm, out_hbm,
                    srcb0, srcb1, wb0, wb1, dstb, rows0, rows1,
                    acc, gsem0, gsem1, isem0, isem1, dsem, ssem0, ssem1):
    sid = lax.axis_index("s")

    def zrow(i, c):
        for k in range(D // 16):
            rows0[i, pl.ds(k * 16, 16)] = jnp.zeros((16,), jnp.float32)
        return c
    lax.fori_loop(0, B, zrow, 0)

    def zcopy(q, c):
        pltpu.sync_copy(rows0, acc.at[pl.ds(sid * RPT + q * B, B)])
        return c
    lax.fori_loop(0, RPT // B, zcopy, 0)
    plsc.subcore_barrier()

    srcb = (srcb0, srcb1)
    wb = (wb0, wb1)
    rows = (rows0, rows1)
    gsem = (gsem0, gsem1)
    isem = (isem0, isem1)
    ssem = (ssem0, ssem1)
    tb = sid * NCHUNK * B

    def sw_issue(j, p):
        base = tb + j * B
        pltpu.async_copy(src_hbm.at[pl.ds(base, B)], srcb[p], isem[p])
        pltpu.async_copy(w_hbm.at[pl.ds(base, B)], wb[p], isem[p])

    def sw_wait(j, p):
        base = tb + j * B
        pltpu.make_async_copy(src_hbm.at[pl.ds(base, B)], srcb[p],
                              isem[p]).wait()
        pltpu.make_async_copy(w_hbm.at[pl.ds(base, B)], wb[p],
                              isem[p]).wait()

    def d_issue(j, q):
        pltpu.async_copy(dst_hbm.at[pl.ds(tb + j * B, B)], dstb.at[q], dsem)

    def d_wait(j, q):
        pltpu.make_async_copy(dst_hbm.at[pl.ds(tb + j * B, B)], dstb.at[q],
                              dsem).wait()

    def s_wait(buf, q):
        pltpu.make_async_copy(rows[buf], acc.at[dstb.at[q]],
                              ssem[buf]).wait()

    def proc(j, buf, pf_gather=True, pf_sw=True, pf_d=True, wait_sprev=True):
        q = lax.rem(j, 3)
        qp = lax.rem(j + 2, 3)
        if wait_sprev:
            s_wait(1 - buf, qp)
        if pf_gather:
            sw_wait(j + 1, 1 - buf)
            pltpu.async_copy(h_hbm.at[srcb[1 - buf]], rows[1 - buf],
                             gsem[1 - buf])
        pltpu.make_async_copy(h_hbm.at[srcb[buf]], rows[buf],
                              gsem[buf]).wait()
        d_wait(j, q)

        def gbody(g, c2):
            wv = wb[buf][pl.ds(g * 16, 16)]
            for l in range(16):
                ws = jnp.broadcast_to(wv[l:l + 1], (16,))
                r = g * 16 + l
                for k in range(D // 16):
                    sl = pl.ds(k * 16, 16)
                    rows[buf][r, sl] = rows[buf][r, sl] * ws
            return c2
        lax.fori_loop(0, B // 16, gbody, 0)

        if pf_sw:
            sw_issue(j + 2, buf)
        pltpu.async_copy(rows[buf], acc.at[dstb.at[q]], ssem[buf], add=True)
        if pf_d:
            d_issue(j + 2, qp)

    pltpu.sync_copy(src_hbm.at[pl.ds(tb, B)], srcb0)
    pltpu.sync_copy(w_hbm.at[pl.ds(tb, B)], wb0)
    d_issue(0, 0)
    d_issue(1, 1)
    pltpu.async_copy(h_hbm.at[srcb0], rows0, gsem0)
    sw_issue(1, 1)

    proc(0, 0, wait_sprev=False)
    proc(1, 1)

    def outer(j2, c):
        j = j2 * 2 + 2
        proc(j, 0)
        proc(j + 1, 1)
        return c
    lax.fori_loop(0, (NCHUNK - 4) // 2, outer, 0)
    proc(NCHUNK - 2, 0, pf_sw=False, pf_d=False)
    proc(NCHUNK - 1, 1, pf_gather=False, pf_sw=False, pf_d=False)
    s_wait(1, (NCHUNK - 1) % 3)

    plsc.subcore_barrier()
    pltpu.sync_copy(acc.at[pl.ds(sid * RPT, RPT)],
                    out_hbm.at[pl.ds(sid * RPT, RPT)])


_sc_conv = functools.partial(
    pl.kernel,
    out_type=jax.ShapeDtypeStruct((NP, D), jnp.float32),
    mesh=plsc.VectorSubcoreMesh(core_axis_name="c", subcore_axis_name="s",
                                num_cores=1),
    scratch_types=[
        pltpu.VMEM((B,), jnp.int32),
        pltpu.VMEM((B,), jnp.int32),
        pltpu.VMEM((B,), jnp.float32),
        pltpu.VMEM((B,), jnp.float32),
        pltpu.VMEM((3, B), jnp.int32),
        pltpu.VMEM((B, D), jnp.float32),
        pltpu.VMEM((B, D), jnp.float32),
        pltpu.VMEM_SHARED((NP, D), jnp.float32),
        pltpu.SemaphoreType.DMA,
        pltpu.SemaphoreType.DMA,
        pltpu.SemaphoreType.DMA,
        pltpu.SemaphoreType.DMA,
        pltpu.SemaphoreType.DMA,
        pltpu.SemaphoreType.DMA,
        pltpu.SemaphoreType.DMA,
    ],
)(_sc_conv_kernel)


def kernel(hidden, in_edge_index, in_edge_weight, out_edge_index,
           out_edge_weight, W_in, b_in, W_out, b_out, W_ih, b_ih, W_hh, b_hh):
    hidden = hidden.astype(jnp.float32)

    h_in, h_out, gh = _phase_a(
        hidden, W_in.T, b_in.reshape(1, D), W_out.T, b_out.reshape(1, D),
        W_hh.T, b_hh.reshape(1, 3 * D))

    def shape_edges(ei, ew):
        return (ei[0].astype(jnp.int32), ei[1].astype(jnp.int32),
                ew.astype(jnp.float32))

    src_i, dst_i, w_i = shape_edges(in_edge_index, in_edge_weight)
    src_o, dst_o, w_o = shape_edges(out_edge_index, out_edge_weight)

    agg_in = _sc_conv(h_in, src_i, dst_i, w_i)
    h_out, _ = lax.optimization_barrier((h_out, agg_in))
    agg_out = _sc_conv(h_out, src_o, dst_o, w_o)

    W_ihT = W_ih.T
    return _phase_c(agg_in[:N], agg_out[:N], hidden, gh,
                    W_ihT[:D], W_ihT[D:], b_ih.reshape(1, 3 * D))
